# Initial kernel scaffold; baseline (speedup 1.0000x reference)
import jax
import jax.numpy as jnp
from jax.experimental import pallas as pl


def kernel(list_neigh, Imagetype_map, atom_type, ImageDR, nghost, c_param_2, c_param_3, W1, b1, W2, b2):
    raise NotImplementedError("write your pallas kernel here")


# speedup vs baseline: 2.8802x; 2.8802x over previous
import functools

import jax
import jax.numpy as jnp
from jax.experimental import pallas as pl

_NTYPES = 2
_MAX_NEIGH = 64
_M = _NTYPES * _MAX_NEIGH
_NMR = 5
_NBR = 9
_NMA = 5
_NBA = 9
_RC_R = 6.0
_RC_A = 4.0
_NEURON = 30
_FEAT = 15

_C3B = (0.238732414637843, 0.119366207318922, 0.119366207318922,
        0.099471839432435, 0.596831036594608, 0.596831036594608,
        0.149207759148652, 0.149207759148652)


def _cheb_terms(rr, rc, nbase):
    fc = jnp.where(rr < rc, 0.5 * (jnp.cos(jnp.pi * rr / rc) + 1.0), 0.0)
    x = 2.0 * (rr / rc - 1.0) ** 2 - 1.0
    ts = [jnp.ones_like(x), x]
    for _ in range(2, nbase):
        ts.append(2.0 * x * ts[-1] - ts[-2])
    return [0.5 * (t + 1.0) * fc for t in ts]


def _nep_body(ln_ref, r_ref, dx_ref, dy_ref, dz_ref, it_ref,
              c2_ref, c3_ref, w1_ref, b1_ref, w2_ref, b2_ref,
              ei_ref, etot_ref):
    r = r_ref[...]
    valid = (ln_ref[...] > 0).astype(jnp.float32)
    dx = dx_ref[...]
    dy = dy_ref[...]
    dz = dz_ref[...]
    inv = 1.0 / jnp.maximum(jnp.sqrt(dx * dx + dy * dy + dz * dz), 1e-8)
    ux = dx * inv
    uy = dy * inv
    uz = dz * inv
    is0 = it_ref[...] == 0

    fkr = _cheb_terms(r, _RC_R, _NBR)
    fka = _cheb_terms(r, _RC_A, _NBA)

    feat_cols = []
    for q in range(_NMR):
        acc0 = fkr[0] * c2_ref[q * _NBR, :][None, :]
        acc1 = fkr[0] * c2_ref[_NMR * _NBR + q * _NBR, :][None, :]
        for k in range(1, _NBR):
            acc0 = acc0 + fkr[k] * c2_ref[q * _NBR + k, :][None, :]
            acc1 = acc1 + fkr[k] * c2_ref[_NMR * _NBR + q * _NBR + k, :][None, :]
        s0 = jnp.sum(acc0 * valid, axis=1, keepdims=True)
        s1 = jnp.sum(acc1 * valid, axis=1, keepdims=True)
        feat_cols.append(jnp.where(is0, s0, s1))

    blm = (uz, ux, uy,
           3.0 * uz * uz - 1.0, ux * uz, uy * uz,
           ux * ux - uy * uy, ux * uy)
    q1_cols = []
    q2_cols = []
    for q in range(_NMA):
        a0 = fka[0] * c3_ref[q * _NBA, :][None, :]
        a1 = fka[0] * c3_ref[_NMA * _NBA + q * _NBA, :][None, :]
        for k in range(1, _NBA):
            a0 = a0 + fka[k] * c3_ref[q * _NBA + k, :][None, :]
            a1 = a1 + fka[k] * c3_ref[_NMA * _NBA + q * _NBA + k, :][None, :]
        a0 = a0 * valid
        a1 = a1 * valid
        q1 = None
        q2 = None
        for p in range(8):
            t0 = jnp.sum(a0 * blm[p], axis=1, keepdims=True)
            t1 = jnp.sum(a1 * blm[p], axis=1, keepdims=True)
            s = jnp.where(is0, t0, t1)
            term = _C3B[p] * s * s
            if p < 3:
                q1 = term if q1 is None else q1 + term
            else:
                q2 = term if q2 is None else q2 + term
        q1_cols.append(q1)
        q2_cols.append(q2)

    feat = jnp.concatenate(feat_cols + q1_cols + q2_cols, axis=1)

    w1 = w1_ref[...]
    h0 = jnp.tanh(
        jnp.dot(feat, w1[0:_FEAT, :], preferred_element_type=jnp.float32)
        + b1_ref[0:1, :])
    h1 = jnp.tanh(
        jnp.dot(feat, w1[_FEAT:2 * _FEAT, :], preferred_element_type=jnp.float32)
        + b1_ref[1:2, :])
    w2 = w2_ref[...]
    e0 = jnp.dot(h0, w2[:, 0:1], preferred_element_type=jnp.float32) + b2_ref[:, 0:1]
    e1 = jnp.dot(h1, w2[:, 1:2], preferred_element_type=jnp.float32) + b2_ref[:, 1:2]
    ei = jnp.where(is0, e0, e1)
    ei_ref[...] = ei

    @pl.when(pl.program_id(0) == 0)
    def _init():
        etot_ref[...] = jnp.zeros_like(etot_ref)

    etot_ref[...] += jnp.sum(ei, keepdims=True)


@functools.partial(jax.jit, static_argnames=("interpret",))
def _nep_forward(list_neigh, Imagetype_map, ImageDR, c_param_2, c_param_3,
                 W1, b1, W2, b2, interpret=False):
    B, N, M = list_neigh.shape
    ln = list_neigh.reshape(N, M)
    r = ImageDR[..., 0].reshape(N, M)
    dx = ImageDR[..., 1].reshape(N, M)
    dy = ImageDR[..., 2].reshape(N, M)
    dz = ImageDR[..., 3].reshape(N, M)
    itype = Imagetype_map.astype(jnp.int32).reshape(N, 1)

    jt = jnp.repeat(jnp.arange(_NTYPES), _MAX_NEIGH)
    c2l = c_param_2[:, jt, :, :].transpose(0, 2, 3, 1).reshape(
        _NTYPES * _NMR * _NBR, M)
    c3l = c_param_3[:, jt, :, :].transpose(0, 2, 3, 1).reshape(
        _NTYPES * _NMA * _NBA, M)
    w1r = W1.reshape(_NTYPES * _FEAT, _NEURON)
    w2r = W2.reshape(_NTYPES, _NEURON).T
    b2r = b2.reshape(1, _NTYPES)

    bn = 400
    grid = (N // bn,)
    row_spec = pl.BlockSpec((bn, M), lambda i: (i, 0))
    const = lambda shape: pl.BlockSpec(shape, lambda i: tuple(0 for _ in shape))

    ei, etot = pl.pallas_call(
        _nep_body,
        grid=grid,
        in_specs=[
            row_spec,
            row_spec, row_spec, row_spec, row_spec,
            pl.BlockSpec((bn, 1), lambda i: (i, 0)),
            const(c2l.shape), const(c3l.shape),
            const(w1r.shape), const(b1.shape), const(w2r.shape), const(b2r.shape),
        ],
        out_specs=[
            pl.BlockSpec((bn, 1), lambda i: (i, 0)),
            pl.BlockSpec((1, 1), lambda i: (0, 0)),
        ],
        out_shape=[
            jax.ShapeDtypeStruct((N, 1), jnp.float32),
            jax.ShapeDtypeStruct((1, 1), jnp.float32),
        ],
        interpret=interpret,
    )(ln, r, dx, dy, dz, itype, c2l, c3l, w1r, b1, w2r, b2r)

    return etot.reshape(B, 1), ei.reshape(B, N)


def kernel(list_neigh, Imagetype_map, atom_type, ImageDR, nghost,
           c_param_2, c_param_3, W1, b1, W2, b2):
    return _nep_forward(list_neigh, Imagetype_map, ImageDR,
                        c_param_2, c_param_3, W1, b1, W2, b2)
